# Initial kernel scaffold; baseline (speedup 1.0000x reference)
import jax
import jax.numpy as jnp
from jax.experimental import pallas as pl


def kernel(t, abars):
    raise NotImplementedError("write your pallas kernel here")


# speedup vs baseline: 38.8410x; 38.8410x over previous
import functools

import jax
import jax.numpy as jnp
from jax import lax
from jax.experimental import pallas as pl
from jax.experimental.pallas import tpu as pltpu, tpu_sc as plsc

T_MAX = 1000
BETA_MIN = 0.0001
BETA_MAX = 0.02

_L = 16
_NC = 2
_NS = 16
_NW = _NC * _NS


def _sc_kernel(B):
    b_per_w = B // _NW
    mesh = plsc.VectorSubcoreMesh(core_axis_name="c", subcore_axis_name="s")

    @functools.partial(
        pl.kernel,
        mesh=mesh,
        out_type=(
            jax.ShapeDtypeStruct((B,), jnp.float32),
            jax.ShapeDtypeStruct((B,), jnp.float32),
        ),
        scratch_types=[
            pltpu.VMEM((b_per_w,), jnp.int32),
            pltpu.VMEM((T_MAX,), jnp.float32),
            pltpu.VMEM((b_per_w,), jnp.float32),
            pltpu.VMEM((b_per_w,), jnp.float32),
        ],
        compiler_params=pltpu.CompilerParams(needs_layout_passes=False),
    )
    def k(t_hbm, abars_hbm, abars_t_hbm, betas_hbm, idx_v, tab_v, oa_v, ob_v):
        wid = lax.axis_index("s") * _NC + lax.axis_index("c")
        base = wid * b_per_w
        pltpu.sync_copy(t_hbm.at[pl.ds(base, b_per_w)], idx_v)
        pltpu.sync_copy(abars_hbm, tab_v)
        scale = jnp.float32((BETA_MAX - BETA_MIN) / T_MAX)
        for j in range(b_per_w // _L):
            tv = idx_v[pl.ds(j * _L, _L)]
            oa_v[pl.ds(j * _L, _L)] = plsc.load_gather(tab_v, [tv])
            ob_v[pl.ds(j * _L, _L)] = tv.astype(jnp.float32) * scale + jnp.float32(BETA_MIN)
        pltpu.sync_copy(oa_v, abars_t_hbm.at[pl.ds(base, b_per_w)])
        pltpu.sync_copy(ob_v, betas_hbm.at[pl.ds(base, b_per_w)])

    return k


def kernel(t, abars):
    B = t.shape[0]
    abars_t, betas = _sc_kernel(B)(t, abars)
    return (abars_t, betas)


# speedup vs baseline: 40.3970x; 1.0401x over previous
import functools

import jax
import jax.numpy as jnp
from jax import lax
from jax.experimental import pallas as pl
from jax.experimental.pallas import tpu as pltpu, tpu_sc as plsc

T_MAX = 1000
BETA_MIN = 0.0001
BETA_MAX = 0.02

_L = 16
_NC = 2
_NS = 16
_NW = _NC * _NS


def _sc_kernel(B):
    b_per_w = B // _NW
    mesh = plsc.VectorSubcoreMesh(core_axis_name="c", subcore_axis_name="s")

    @functools.partial(
        pl.kernel,
        mesh=mesh,
        out_type=(
            jax.ShapeDtypeStruct((B,), jnp.float32),
            jax.ShapeDtypeStruct((B,), jnp.float32),
        ),
        scratch_types=[
            pltpu.VMEM((b_per_w,), jnp.int32),
            pltpu.VMEM((T_MAX,), jnp.float32),
            pltpu.VMEM((b_per_w,), jnp.float32),
            pltpu.VMEM((b_per_w,), jnp.float32),
            pltpu.SemaphoreType.DMA,
            pltpu.SemaphoreType.DMA,
        ],
        compiler_params=pltpu.CompilerParams(
            needs_layout_passes=False,
            skip_device_barrier=True,
            disable_bounds_checks=True,
            disable_semaphore_checks=True,
        ),
    )
    def k(t_hbm, abars_hbm, abars_t_hbm, betas_hbm, idx_v, tab_v, oa_v, ob_v,
          sem0, sem1):
        wid = lax.axis_index("s") * _NC + lax.axis_index("c")
        base = wid * b_per_w
        cp_idx = pltpu.async_copy(t_hbm.at[pl.ds(base, b_per_w)], idx_v, sem0)
        cp_tab = pltpu.async_copy(abars_hbm, tab_v, sem1)
        scale = jnp.float32((BETA_MAX - BETA_MIN) / T_MAX)
        cp_idx.wait()
        for j in range(b_per_w // _L):
            tv = idx_v[pl.ds(j * _L, _L)]
            ob_v[pl.ds(j * _L, _L)] = tv.astype(jnp.float32) * scale + jnp.float32(BETA_MIN)
        cp_tab.wait()
        for j in range(b_per_w // _L):
            tv = idx_v[pl.ds(j * _L, _L)]
            oa_v[pl.ds(j * _L, _L)] = plsc.load_gather(tab_v, [tv])
        cp_a = pltpu.async_copy(oa_v, abars_t_hbm.at[pl.ds(base, b_per_w)], sem0)
        cp_b = pltpu.async_copy(ob_v, betas_hbm.at[pl.ds(base, b_per_w)], sem1)
        cp_a.wait()
        cp_b.wait()

    return k


def kernel(t, abars):
    B = t.shape[0]
    abars_t, betas = _sc_kernel(B)(t, abars)
    return (abars_t, betas)


# speedup vs baseline: 44.0652x; 1.0908x over previous
import functools

import jax
import jax.numpy as jnp
from jax import lax
from jax.experimental import pallas as pl
from jax.experimental.pallas import tpu as pltpu, tpu_sc as plsc

T_MAX = 1000
BETA_MIN = 0.0001
BETA_MAX = 0.02

_L = 16
_NC = 1
_NS = 16
_NW = _NC * _NS


def _sc_kernel(B):
    b_per_w = B // _NW
    mesh = plsc.VectorSubcoreMesh(core_axis_name="c", subcore_axis_name="s", num_cores=_NC)

    @functools.partial(
        pl.kernel,
        mesh=mesh,
        out_type=(
            jax.ShapeDtypeStruct((B,), jnp.float32),
            jax.ShapeDtypeStruct((B,), jnp.float32),
        ),
        scratch_types=[
            pltpu.VMEM((b_per_w,), jnp.int32),
            pltpu.VMEM((T_MAX,), jnp.float32),
            pltpu.VMEM((b_per_w,), jnp.float32),
            pltpu.VMEM((b_per_w,), jnp.float32),
            pltpu.SemaphoreType.DMA,
            pltpu.SemaphoreType.DMA,
        ],
        compiler_params=pltpu.CompilerParams(
            needs_layout_passes=False,
            skip_device_barrier=True,
            disable_bounds_checks=True,
            disable_semaphore_checks=True,
        ),
    )
    def k(t_hbm, abars_hbm, abars_t_hbm, betas_hbm, idx_v, tab_v, oa_v, ob_v,
          sem0, sem1):
        wid = lax.axis_index("s") * _NC + lax.axis_index("c") if _NC > 1 else lax.axis_index("s")
        base = wid * b_per_w
        cp_idx = pltpu.async_copy(t_hbm.at[pl.ds(base, b_per_w)], idx_v, sem0)
        cp_tab = pltpu.async_copy(abars_hbm, tab_v, sem1)
        scale = jnp.float32((BETA_MAX - BETA_MIN) / T_MAX)
        bmin = jnp.float32(BETA_MIN)
        cp_idx.wait()
        cp_tab.wait()

        U = 4
        def body(i, carry):
            off = i * (_L * U)
            for u in range(U):
                o = off + u * _L
                tv = idx_v[pl.ds(o, _L)]
                oa_v[pl.ds(o, _L)] = plsc.load_gather(tab_v, [tv])
                ob_v[pl.ds(o, _L)] = tv.astype(jnp.float32) * scale + bmin
            return carry
        lax.fori_loop(0, b_per_w // (_L * U), body, 0)

        cp_a = pltpu.async_copy(oa_v, abars_t_hbm.at[pl.ds(base, b_per_w)], sem0)
        cp_b = pltpu.async_copy(ob_v, betas_hbm.at[pl.ds(base, b_per_w)], sem1)
        cp_a.wait()
        cp_b.wait()

    return k


def kernel(t, abars):
    B = t.shape[0]
    abars_t, betas = _sc_kernel(B)(t, abars)
    return (abars_t, betas)


# speedup vs baseline: 44.4006x; 1.0076x over previous
import functools

import jax
import jax.numpy as jnp
from jax import lax
from jax.experimental import pallas as pl
from jax.experimental.pallas import tpu as pltpu, tpu_sc as plsc

T_MAX = 1000
BETA_MIN = 0.0001
BETA_MAX = 0.02

_L = 16
_NC = 1
_NS = 16
_NW = _NC * _NS


def _sc_kernel(B):
    b_per_w = B // _NW
    mesh = plsc.VectorSubcoreMesh(core_axis_name="c", subcore_axis_name="s", num_cores=_NC)

    @functools.partial(
        pl.kernel,
        mesh=mesh,
        out_type=(
            jax.ShapeDtypeStruct((B,), jnp.float32),
            jax.ShapeDtypeStruct((B,), jnp.float32),
        ),
        scratch_types=[
            pltpu.VMEM((b_per_w,), jnp.int32),
            pltpu.VMEM((T_MAX,), jnp.float32),
            pltpu.VMEM((b_per_w,), jnp.float32),
            pltpu.VMEM((b_per_w,), jnp.float32),
            pltpu.SemaphoreType.DMA,
            pltpu.SemaphoreType.DMA,
        ],
        compiler_params=pltpu.CompilerParams(
            needs_layout_passes=False,
            skip_device_barrier=True,
            disable_bounds_checks=True,
            disable_semaphore_checks=True,
        ),
    )
    def k(t_hbm, abars_hbm, abars_t_hbm, betas_hbm, idx_v, tab_v, oa_v, ob_v,
          sem0, sem1):
        wid = lax.axis_index("s") * _NC + lax.axis_index("c") if _NC > 1 else lax.axis_index("s")
        base = wid * b_per_w
        cp_idx = pltpu.async_copy(t_hbm.at[pl.ds(base, b_per_w)], idx_v, sem0)
        cp_tab = pltpu.async_copy(abars_hbm, tab_v, sem1)
        scale = jnp.float32((BETA_MAX - BETA_MIN) / T_MAX)
        bmin = jnp.float32(BETA_MIN)
        cp_idx.wait()
        cp_tab.wait()

        @plsc.parallel_loop(0, b_per_w, _L, unroll=4)
        def body(o):
            tv = idx_v[pl.ds(o, _L)]
            oa_v[pl.ds(o, _L)] = plsc.load_gather(tab_v, [tv])
            ob_v[pl.ds(o, _L)] = tv.astype(jnp.float32) * scale + bmin

        cp_a = pltpu.async_copy(oa_v, abars_t_hbm.at[pl.ds(base, b_per_w)], sem0)
        cp_b = pltpu.async_copy(ob_v, betas_hbm.at[pl.ds(base, b_per_w)], sem1)
        cp_a.wait()
        cp_b.wait()

    return k


def kernel(t, abars):
    B = t.shape[0]
    abars_t, betas = _sc_kernel(B)(t, abars)
    return (abars_t, betas)
